# Initial kernel scaffold; baseline (speedup 1.0000x reference)
import jax
import jax.numpy as jnp
from jax.experimental import pallas as pl


def kernel(x, edge_index, W_att, a_src, a_dst, W_state, W_out):
    raise NotImplementedError("write your pallas kernel here")


# speedup vs baseline: 138.3197x; 138.3197x over previous
import functools

import jax
import jax.numpy as jnp
from jax import lax
from jax.experimental import pallas as pl
from jax.experimental.pallas import tpu as pltpu
from jax.experimental.pallas import tpu_sc as plsc

_NC = 2
_NS = 16
_NW = _NC * _NS
_L = 16


def _pre_body(xt_ref, asdT_ref, WattT_ref, WoutT_ref, WstateT_ref, out_ref):
    Np = xt_ref.shape[1]
    wsdT = jnp.dot(asdT_ref[...], WattT_ref[...], preferred_element_type=jnp.float32)
    sd_t = jnp.dot(wsdT, xt_ref[...], preferred_element_type=jnp.float32)
    PT = jnp.dot(WoutT_ref[...], WstateT_ref[...], preferred_element_type=jnp.float32)
    mp_t = jnp.dot(PT, xt_ref[0:2, :], preferred_element_type=jnp.float32)
    c = jnp.maximum(jnp.max(sd_t[0:1, :]) + jnp.max(sd_t[1:2, :]), 0.0)
    out_ref[0:2, :] = sd_t
    out_ref[2:4, :] = mp_t
    out_ref[4:5, :] = jnp.full((1, Np), c, jnp.float32)
    out_ref[5:8, :] = jnp.zeros((3, Np), jnp.float32)


def _fin_body(den_ref, n0_ref, n1_ref, out_ref):
    den = jnp.sum(den_ref[...], axis=0, keepdims=True) + 1e-16
    out_ref[0:1, :] = jnp.tanh(jnp.sum(n0_ref[...], axis=0, keepdims=True) / den)
    out_ref[1:2, :] = jnp.tanh(jnp.sum(n1_ref[...], axis=0, keepdims=True) / den)


def _make_edge_kernel(Np, Ew):
    mesh = plsc.VectorSubcoreMesh(core_axis_name="c", subcore_axis_name="s")

    @functools.partial(
        pl.kernel,
        mesh=mesh,
        out_type=[jax.ShapeDtypeStruct((_NW, Np), jnp.float32)] * 3,
        compiler_params=pltpu.CompilerParams(needs_layout_passes=False),
        scratch_types=[
            pltpu.VMEM((Np,), jnp.float32),
            pltpu.VMEM((Np,), jnp.float32),
            pltpu.VMEM((Np,), jnp.float32),
            pltpu.VMEM((Np,), jnp.float32),
            pltpu.VMEM((_L,), jnp.float32),
            pltpu.VMEM((Ew,), jnp.int32),
            pltpu.VMEM((Ew,), jnp.int32),
            pltpu.VMEM((Np,), jnp.float32),
            pltpu.VMEM((Np,), jnp.float32),
            pltpu.VMEM((Np,), jnp.float32),
        ],
    )
    def edge_kernel(node_hbm, src_hbm, dst_hbm, den_out, n0_out, n1_out,
                    as_v, ad_v, m0_v, m1_v, c_v, src_v, dst_v,
                    den_v, n0_v, n1_v):
        wid = lax.axis_index("s") * _NC + lax.axis_index("c")
        base = wid * Ew
        pltpu.sync_copy(node_hbm.at[0], as_v)
        pltpu.sync_copy(node_hbm.at[1], ad_v)
        pltpu.sync_copy(node_hbm.at[2], m0_v)
        pltpu.sync_copy(node_hbm.at[3], m1_v)
        pltpu.sync_copy(node_hbm.at[4, pl.ds(0, _L)], c_v)
        pltpu.sync_copy(src_hbm.at[pl.ds(base, Ew)], src_v)
        pltpu.sync_copy(dst_hbm.at[pl.ds(base, Ew)], dst_v)

        zeros = jnp.zeros((_L,), jnp.float32)

        def zbody(j, carry):
            den_v[pl.ds(j * _L, _L)] = zeros
            n0_v[pl.ds(j * _L, _L)] = zeros
            n1_v[pl.ds(j * _L, _L)] = zeros
            return carry

        lax.fori_loop(0, Np // _L, zbody, 0)

        cv = c_v[...]

        def body(j, carry):
            o = j * _L
            isrc = src_v[pl.ds(o, _L)]
            idst = dst_v[pl.ds(o, _L)]
            a_s = plsc.load_gather(as_v, [isrc])
            a_d = plsc.load_gather(ad_v, [idst])
            m0 = plsc.load_gather(m0_v, [isrc])
            m1 = plsc.load_gather(m1_v, [isrc])
            s = a_s + a_d
            e = jnp.where(s >= 0.0, s, s * 0.2)
            w = jnp.exp(e - cv)
            plsc.addupdate_scatter(den_v, [idst], w)
            plsc.addupdate_scatter(n0_v, [idst], w * m0)
            plsc.addupdate_scatter(n1_v, [idst], w * m1)
            return carry

        lax.fori_loop(0, Ew // _L, body, 0)

        pltpu.sync_copy(den_v, den_out.at[wid])
        pltpu.sync_copy(n0_v, n0_out.at[wid])
        pltpu.sync_copy(n1_v, n1_out.at[wid])

    return edge_kernel


def kernel(x, edge_index, W_att, a_src, a_dst, W_state, W_out):
    N = x.shape[0]
    E = edge_index.shape[1]
    Np = ((N + 127) // 128) * 128
    Ew = E // _NW

    xt = jnp.zeros((4, Np), jnp.float32).at[:, :N].set(x.T)
    asdT = jnp.stack([a_src, a_dst], axis=0)
    WattT = W_att.T
    WoutT = W_out.T
    WstateT = W_state.T

    node_t = pl.pallas_call(
        _pre_body,
        out_shape=jax.ShapeDtypeStruct((8, Np), jnp.float32),
    )(xt, asdT, WattT, WoutT, WstateT)

    src = edge_index[0]
    dst = edge_index[1]
    den_p, n0_p, n1_p = _make_edge_kernel(Np, Ew)(node_t, src, dst)

    out3 = pl.pallas_call(
        _fin_body,
        out_shape=jax.ShapeDtypeStruct((2, Np), jnp.float32),
    )(den_p, n0_p, n1_p)

    return out3[:, :N].T.ravel()


# speedup vs baseline: 153.1121x; 1.1069x over previous
import functools

import jax
import jax.numpy as jnp
from jax import lax
from jax.experimental import pallas as pl
from jax.experimental.pallas import tpu as pltpu
from jax.experimental.pallas import tpu_sc as plsc

_NC = 2
_NS = 16
_NW = _NC * _NS
_L = 16


def _pre_body(x_ref, asd_ref, Watt_ref, Wstate_ref, Wout_ref, tab_ref, c_ref):
    wsd = jnp.dot(Watt_ref[...], asd_ref[...], preferred_element_type=jnp.float32)
    P = jnp.dot(Wstate_ref[...], Wout_ref[...], preferred_element_type=jnp.float32)
    sd = jnp.dot(x_ref[...], wsd, preferred_element_type=jnp.float32)
    mp = jnp.dot(x_ref[:, 0:2], P, preferred_element_type=jnp.float32)
    tab_ref[...] = jnp.concatenate([sd, mp], axis=1)
    c = jnp.maximum(jnp.max(sd[:, 0:1]) + jnp.max(sd[:, 1:2]), 0.0)
    c_ref[...] = jnp.full((1, _L), c, jnp.float32)


def _fin_body(den_ref, n0_ref, n1_ref, out_ref):
    den = jnp.sum(den_ref[...], axis=0, keepdims=True) + 1e-16
    out_ref[0:1, :] = jnp.tanh(jnp.sum(n0_ref[...], axis=0, keepdims=True) / den)
    out_ref[1:2, :] = jnp.tanh(jnp.sum(n1_ref[...], axis=0, keepdims=True) / den)


def _make_edge_kernel(N, Np, Ew):
    mesh = plsc.VectorSubcoreMesh(core_axis_name="c", subcore_axis_name="s")

    @functools.partial(
        pl.kernel,
        mesh=mesh,
        out_type=[jax.ShapeDtypeStruct((_NW, Np), jnp.float32)] * 3,
        compiler_params=pltpu.CompilerParams(needs_layout_passes=False),
        scratch_types=[
            pltpu.VMEM((N * 4,), jnp.float32),
            pltpu.VMEM((_L,), jnp.float32),
            pltpu.VMEM((Ew,), jnp.int32),
            pltpu.VMEM((Ew,), jnp.int32),
            pltpu.VMEM((Np,), jnp.float32),
            pltpu.VMEM((Np,), jnp.float32),
            pltpu.VMEM((Np,), jnp.float32),
        ],
    )
    def edge_kernel(tab_hbm, c_hbm, src_hbm, dst_hbm, den_out, n0_out, n1_out,
                    tab_v, c_v, src_v, dst_v, den_v, n0_v, n1_v):
        wid = lax.axis_index("s") * _NC + lax.axis_index("c")
        base = wid * Ew
        pltpu.sync_copy(tab_hbm, tab_v)
        pltpu.sync_copy(c_hbm, c_v)
        pltpu.sync_copy(src_hbm.at[pl.ds(base, Ew)], src_v)
        pltpu.sync_copy(dst_hbm.at[pl.ds(base, Ew)], dst_v)

        zeros = jnp.zeros((_L,), jnp.float32)

        @plsc.parallel_loop(0, Np // _L, unroll=8)
        def _(j):
            o = j * _L
            den_v[pl.ds(o, _L)] = zeros
            n0_v[pl.ds(o, _L)] = zeros
            n1_v[pl.ds(o, _L)] = zeros

        cv = c_v[...]

        @plsc.parallel_loop(0, Ew // _L, unroll=4)
        def _(j):
            o = j * _L
            i4s = src_v[pl.ds(o, _L)] * 4
            idst = dst_v[pl.ds(o, _L)]
            a_s = plsc.load_gather(tab_v, [i4s])
            a_d = plsc.load_gather(tab_v, [idst * 4 + 1])
            m0 = plsc.load_gather(tab_v, [i4s + 2])
            m1 = plsc.load_gather(tab_v, [i4s + 3])
            s = a_s + a_d
            e = jnp.where(s >= 0.0, s, s * 0.2)
            w = jnp.exp(e - cv)
            plsc.addupdate_scatter(den_v, [idst], w)
            plsc.addupdate_scatter(n0_v, [idst], w * m0)
            plsc.addupdate_scatter(n1_v, [idst], w * m1)

        pltpu.sync_copy(den_v, den_out.at[wid])
        pltpu.sync_copy(n0_v, n0_out.at[wid])
        pltpu.sync_copy(n1_v, n1_out.at[wid])

    return edge_kernel


def kernel(x, edge_index, W_att, a_src, a_dst, W_state, W_out):
    N = x.shape[0]
    E = edge_index.shape[1]
    Np = ((N + 127) // 128) * 128
    Ew = E // _NW

    asd = jnp.stack([a_src, a_dst], axis=1)

    tab, c16 = pl.pallas_call(
        _pre_body,
        out_shape=[
            jax.ShapeDtypeStruct((N, 4), jnp.float32),
            jax.ShapeDtypeStruct((1, _L), jnp.float32),
        ],
    )(x, asd, W_att, W_state, W_out)

    src = edge_index[0]
    dst = edge_index[1]
    den_p, n0_p, n1_p = _make_edge_kernel(N, Np, Ew)(
        tab.reshape(-1), c16.reshape(-1), src, dst)

    out3 = pl.pallas_call(
        _fin_body,
        out_shape=jax.ShapeDtypeStruct((2, Np), jnp.float32),
    )(den_p, n0_p, n1_p)

    return out3[:, :N].T.ravel()
